# baseline (speedup 1.0000x reference)
import functools

import jax
import jax.numpy as jnp
from jax import lax
from jax.experimental import pallas as pl
from jax.experimental.pallas import tpu as pltpu
from jax.experimental.pallas import tpu_sc as plsc

_NC, _NS, _L = 2, 16, 16


@functools.lru_cache(maxsize=None)
def _make_sc(B, S, D):
    R = B * S
    NW = _NC * _NS
    rows_w = R // NW
    wpb = S // rows_w
    CH = 16
    n_chunks = rows_w // CH
    CE = CH * D
    n_vec = CE // _L

    mesh = plsc.VectorSubcoreMesh(core_axis_name="c", subcore_axis_name="s")
    out_t = jax.ShapeDtypeStruct((R * D,), jnp.float32)

    @functools.partial(
        pl.kernel,
        mesh=mesh,
        out_type=[out_t, out_t, out_t],
        scratch_types=[pltpu.VMEM((CE,), jnp.float32) for _ in range(4)],
    )
    def k(r_hbm, s_hbm, a_hbm, p_hbm, o0, o1, o2, pv, x0, x1, x2):
        wid = lax.axis_index("s") * _NC + lax.axis_index("c")
        e0_w = wid * rows_w * D
        pe_w = (wid % wpb) * rows_w * D

        def chunk(ci, _):
            e0 = e0_w + ci * CE
            pe = pe_w + ci * CE
            pltpu.sync_copy(p_hbm.at[pl.ds(pe, CE)], pv)
            pltpu.sync_copy(r_hbm.at[pl.ds(e0, CE)], x0)
            pltpu.sync_copy(s_hbm.at[pl.ds(e0, CE)], x1)
            pltpu.sync_copy(a_hbm.at[pl.ds(e0, CE)], x2)

            def body(i, _):
                off = i * _L
                pval = pv[pl.ds(off, _L)]
                plsc.addupdate(x0.at[pl.ds(off, _L)], pval)
                plsc.addupdate(x1.at[pl.ds(off, _L)], pval)
                plsc.addupdate(x2.at[pl.ds(off, _L)], pval)
                return 0

            lax.fori_loop(0, n_vec, body, 0)
            pltpu.sync_copy(x0, o0.at[pl.ds(e0, CE)])
            pltpu.sync_copy(x1, o1.at[pl.ds(e0, CE)])
            pltpu.sync_copy(x2, o2.at[pl.ds(e0, CE)])
            return 0

        lax.fori_loop(0, n_chunks, chunk, 0)

    return k


def kernel(rtg, state, action, pos_table):
    B, S, D = rtg.shape
    k = _make_sc(B, S, D)
    o0, o1, o2 = k(
        rtg.reshape(-1),
        state.reshape(-1),
        action.reshape(-1),
        pos_table[:S].reshape(-1),
    )
    shp = (B, S, D)
    return (o0.reshape(shp), o1.reshape(shp), o2.reshape(shp))


# speedup vs baseline: 1.3371x; 1.3371x over previous
import functools

import jax
import jax.numpy as jnp
from jax import lax
from jax.experimental import pallas as pl
from jax.experimental.pallas import tpu as pltpu
from jax.experimental.pallas import tpu_sc as plsc

_NC, _NS, _L = 2, 16, 16


@functools.lru_cache(maxsize=None)
def _make_sc(B, S, D):
    NW = _NC * _NS
    wpw = S // NW
    CH = 16
    CE = CH * D
    n_pc = wpw // CH
    n_ph = n_pc * 3 * B
    NB = 4

    mesh = plsc.VectorSubcoreMesh(core_axis_name="c", subcore_axis_name="s")
    out_t = jax.ShapeDtypeStruct((B * S * D,), jnp.float32)

    @functools.partial(
        pl.kernel,
        mesh=mesh,
        out_type=[out_t, out_t, out_t],
        scratch_types=(
            [pltpu.VMEM((CE,), jnp.float32) for _ in range(NB + 2)]
            + [pltpu.SemaphoreType.DMA for _ in range(2 * NB + 2)]
        ),
    )
    def k(r_hbm, s_hbm, a_hbm, p_hbm, o0, o1, o2, *scratch):
        xb = scratch[:NB]
        pv = scratch[NB:NB + 2]
        in_s = scratch[NB + 2:2 * NB + 2]
        out_s = scratch[2 * NB + 2:3 * NB + 2]
        pos_s = scratch[3 * NB + 2:]
        ins = (r_hbm, s_hbm, a_hbm)
        outs = (o0, o1, o2)

        wid = lax.axis_index("s") * _NC + lax.axis_index("c")
        prow0 = wid * wpw

        def pbase(pc):
            return (prow0 + pc * CH) * D

        def ebase(ci):
            pc, r = divmod(ci, 3 * B)
            b = r % B
            return (b * S + prow0 + pc * CH) * D

        def tensor_of(ci):
            return (ci % (3 * B)) // B

        pltpu.async_copy(p_hbm.at[pl.ds(pbase(0), CE)], pv[0], pos_s[0])
        for ci in range(NB - 1):
            pltpu.async_copy(
                ins[tensor_of(ci)].at[pl.ds(ebase(ci), CE)], xb[ci], in_s[ci])

        for ci in range(n_ph):
            pc, q = ci // (3 * B), ci % NB
            if ci % (3 * B) == 0:
                pltpu.make_async_copy(
                    p_hbm.at[pl.ds(0, CE)], pv[pc % 2], pos_s[pc % 2]).wait()
                if pc + 1 < n_pc:
                    pltpu.async_copy(p_hbm.at[pl.ds(pbase(pc + 1), CE)],
                                     pv[(pc + 1) % 2], pos_s[(pc + 1) % 2])
            pltpu.make_async_copy(
                r_hbm.at[pl.ds(0, CE)], xb[q], in_s[q]).wait()

            xq, pvp = xb[q], pv[pc % 2]

            @plsc.parallel_loop(0, CE, step=_L, unroll=8)
            def _(off):
                plsc.addupdate(xq.at[pl.ds(off, _L)], pvp[pl.ds(off, _L)])

            pltpu.async_copy(
                xq, outs[tensor_of(ci)].at[pl.ds(ebase(ci), CE)], out_s[q])

            nci = ci + NB - 1
            if nci < n_ph:
                nq = nci % NB
                if ci >= 1:
                    pltpu.make_async_copy(
                        xb[nq], o0.at[pl.ds(0, CE)], out_s[nq]).wait()
                pltpu.async_copy(
                    ins[tensor_of(nci)].at[pl.ds(ebase(nci), CE)],
                    xb[nq], in_s[nq])

        for ci in range(n_ph - NB, n_ph):
            pltpu.make_async_copy(
                xb[ci % NB], o0.at[pl.ds(0, CE)], out_s[ci % NB]).wait()

    return k


def kernel(rtg, state, action, pos_table):
    B, S, D = rtg.shape
    k = _make_sc(B, S, D)
    o0, o1, o2 = k(
        rtg.reshape(-1),
        state.reshape(-1),
        action.reshape(-1),
        pos_table[:S].reshape(-1),
    )
    shp = (B, S, D)
    return (o0.reshape(shp), o1.reshape(shp), o2.reshape(shp))


# speedup vs baseline: 4.1012x; 3.0672x over previous
import functools

import jax
import jax.numpy as jnp
from jax import lax
from jax.experimental import pallas as pl
from jax.experimental.pallas import tpu as pltpu
from jax.experimental.pallas import tpu_sc as plsc

_NC, _NS, _L = 2, 16, 16


@functools.lru_cache(maxsize=None)
def _make_sc(B, S, D):
    NW = _NC * _NS
    wpw = S // NW
    CH = 16
    CE = CH * D
    n_pc = wpw // CH
    n_ph = n_pc * 3 * B
    NB = 4
    cshift = D.bit_length() - 1
    cmask = (D // _L) - 1

    mesh = plsc.VectorSubcoreMesh(core_axis_name="c", subcore_axis_name="s")
    out_t = jax.ShapeDtypeStruct((B * S, D), jnp.float32)

    @functools.partial(
        pl.kernel,
        mesh=mesh,
        out_type=[out_t, out_t, out_t],
        scratch_types=(
            [pltpu.VMEM((CH, D), jnp.float32) for _ in range(NB + 2)]
            + [pltpu.SemaphoreType.DMA for _ in range(2 * NB + 2)]
        ),
    )
    def k(r_hbm, s_hbm, a_hbm, p_hbm, o0, o1, o2, *scratch):
        xb = scratch[:NB]
        pv = scratch[NB:NB + 2]
        in_s = scratch[NB + 2:2 * NB + 2]
        out_s = scratch[2 * NB + 2:3 * NB + 2]
        pos_s = scratch[3 * NB + 2:]
        ins = (r_hbm, s_hbm, a_hbm)
        outs = (o0, o1, o2)

        wid = lax.axis_index("s") * _NC + lax.axis_index("c")
        prow0 = wid * wpw

        def pbase(pc):
            return prow0 + pc * CH

        def rbase(ci):
            pc, r = divmod(ci, 3 * B)
            b = r % B
            return b * S + prow0 + pc * CH

        def tensor_of(ci):
            return (ci % (3 * B)) // B

        pltpu.async_copy(p_hbm.at[pl.ds(pbase(0), CH)], pv[0], pos_s[0])
        for ci in range(NB - 1):
            pltpu.async_copy(
                ins[tensor_of(ci)].at[pl.ds(rbase(ci), CH)], xb[ci], in_s[ci])

        for ci in range(n_ph):
            pc, q = ci // (3 * B), ci % NB
            if ci % (3 * B) == 0:
                pltpu.make_async_copy(
                    p_hbm.at[pl.ds(0, CH)], pv[pc % 2], pos_s[pc % 2]).wait()
                if pc + 1 < n_pc:
                    pltpu.async_copy(p_hbm.at[pl.ds(pbase(pc + 1), CH)],
                                     pv[(pc + 1) % 2], pos_s[(pc + 1) % 2])
            pltpu.make_async_copy(
                r_hbm.at[pl.ds(0, CH)], xb[q], in_s[q]).wait()

            xq, pvp = xb[q], pv[pc % 2]

            @plsc.parallel_loop(0, CE // _L, unroll=8)
            def _(i):
                r = lax.shift_right_logical(i, cshift - 4)
                c = pl.multiple_of(
                    lax.shift_left(lax.bitwise_and(i, cmask), 4), _L)
                plsc.addupdate(xq.at[r, pl.ds(c, _L)], pvp[r, pl.ds(c, _L)])

            pltpu.async_copy(
                xq, outs[tensor_of(ci)].at[pl.ds(rbase(ci), CH)], out_s[q])

            nci = ci + NB - 1
            if nci < n_ph:
                nq = nci % NB
                if ci >= 1:
                    pltpu.make_async_copy(
                        xb[nq], o0.at[pl.ds(0, CH)], out_s[nq]).wait()
                pltpu.async_copy(
                    ins[tensor_of(nci)].at[pl.ds(rbase(nci), CH)],
                    xb[nq], in_s[nq])

        for ci in range(n_ph - NB, n_ph):
            pltpu.make_async_copy(
                xb[ci % NB], o0.at[pl.ds(0, CH)], out_s[ci % NB]).wait()

    return k


def kernel(rtg, state, action, pos_table):
    B, S, D = rtg.shape
    k = _make_sc(B, S, D)
    o0, o1, o2 = k(
        rtg.reshape(B * S, D),
        state.reshape(B * S, D),
        action.reshape(B * S, D),
        pos_table[:S],
    )
    shp = (B, S, D)
    return (o0.reshape(shp), o1.reshape(shp), o2.reshape(shp))


# speedup vs baseline: 4.5664x; 1.1134x over previous
import functools

import jax
import jax.numpy as jnp
from jax import lax
from jax.experimental import pallas as pl
from jax.experimental.pallas import tpu as pltpu
from jax.experimental.pallas import tpu_sc as plsc

_NC, _NS, _L = 2, 16, 16


@functools.lru_cache(maxsize=None)
def _make_sc(B, S, D, nt):
    NW = _NC * _NS
    wpw = S // NW
    CH = 16
    CE = CH * D
    n_pc = wpw // CH
    n_ph = n_pc * nt * B
    NB = 4
    cshift = D.bit_length() - 1
    cmask = (D // _L) - 1

    mesh = plsc.VectorSubcoreMesh(core_axis_name="c", subcore_axis_name="s")
    out_t = jax.ShapeDtypeStruct((B * S, D), jnp.float32)

    @functools.partial(
        pl.kernel,
        mesh=mesh,
        out_type=[out_t] * nt,
        scratch_types=(
            [pltpu.VMEM((CH, D), jnp.float32) for _ in range(NB + 2)]
            + [pltpu.SemaphoreType.DMA for _ in range(2 * NB + 2)]
        ),
    )
    def k(*args):
        ins = args[:nt]
        p_hbm = args[nt]
        outs = args[nt + 1:2 * nt + 1]
        scratch = args[2 * nt + 1:]
        xb = scratch[:NB]
        pv = scratch[NB:NB + 2]
        in_s = scratch[NB + 2:2 * NB + 2]
        out_s = scratch[2 * NB + 2:3 * NB + 2]
        pos_s = scratch[3 * NB + 2:]

        wid = lax.axis_index("s") * _NC + lax.axis_index("c")
        prow0 = wid * wpw

        def pbase(pc):
            return prow0 + pc * CH

        def rbase(ci):
            pc, r = divmod(ci, nt * B)
            b = r % B
            return b * S + prow0 + pc * CH

        def tensor_of(ci):
            return (ci % (nt * B)) // B

        pltpu.async_copy(p_hbm.at[pl.ds(pbase(0), CH)], pv[0], pos_s[0])
        for ci in range(NB - 1):
            pltpu.async_copy(
                ins[tensor_of(ci)].at[pl.ds(rbase(ci), CH)], xb[ci], in_s[ci])

        for ci in range(n_ph):
            pc, q = ci // (nt * B), ci % NB
            if ci % (nt * B) == 0:
                pltpu.make_async_copy(
                    p_hbm.at[pl.ds(0, CH)], pv[pc % 2], pos_s[pc % 2]).wait()
                if pc + 1 < n_pc:
                    pltpu.async_copy(p_hbm.at[pl.ds(pbase(pc + 1), CH)],
                                     pv[(pc + 1) % 2], pos_s[(pc + 1) % 2])
            pltpu.make_async_copy(
                ins[0].at[pl.ds(0, CH)], xb[q], in_s[q]).wait()

            xq, pvp = xb[q], pv[pc % 2]

            @plsc.parallel_loop(0, CE // _L, unroll=8)
            def _(i):
                r = lax.shift_right_logical(i, cshift - 4)
                c = pl.multiple_of(
                    lax.shift_left(lax.bitwise_and(i, cmask), 4), _L)
                plsc.addupdate(xq.at[r, pl.ds(c, _L)], pvp[r, pl.ds(c, _L)])

            pltpu.async_copy(
                xq, outs[tensor_of(ci)].at[pl.ds(rbase(ci), CH)], out_s[q])

            nci = ci + NB - 1
            if nci < n_ph:
                nq = nci % NB
                if ci >= 1:
                    pltpu.make_async_copy(
                        xb[nq], outs[0].at[pl.ds(0, CH)], out_s[nq]).wait()
                pltpu.async_copy(
                    ins[tensor_of(nci)].at[pl.ds(rbase(nci), CH)],
                    xb[nq], in_s[nq])

        for ci in range(max(n_ph - NB, 0), n_ph):
            pltpu.make_async_copy(
                xb[ci % NB], outs[0].at[pl.ds(0, CH)], out_s[ci % NB]).wait()

    return k


def _tc_body(x0_ref, x1_ref, pos_ref, o0, o1):
    p = pos_ref[...]
    o0[0] = x0_ref[0] + p
    o1[0] = x1_ref[0] + p


@functools.lru_cache(maxsize=None)
def _make_tc(B, S, D):
    BS = 512
    x_spec = pl.BlockSpec((1, BS, D), lambda s, b: (b, s, 0))
    pos_spec = pl.BlockSpec((BS, D), lambda s, b: (s, 0))
    out_shape = jax.ShapeDtypeStruct((B, S, D), jnp.float32)
    return pl.pallas_call(
        _tc_body,
        grid=(S // BS, B),
        in_specs=[x_spec, x_spec, pos_spec],
        out_specs=[x_spec, x_spec],
        out_shape=[out_shape, out_shape],
    )


def kernel(rtg, state, action, pos_table):
    B, S, D = rtg.shape
    pos = pos_table[:S]
    (o2,) = _make_sc(B, S, D, 1)(action.reshape(B * S, D), pos)
    o0, o1 = _make_tc(B, S, D)(rtg, state, pos)
    return (o0, o1, o2.reshape(B, S, D))


# speedup vs baseline: 4.5685x; 1.0005x over previous
import functools

import jax
import jax.numpy as jnp
from jax import lax
from jax.experimental import pallas as pl
from jax.experimental.pallas import tpu as pltpu
from jax.experimental.pallas import tpu_sc as plsc

_NC, _NS, _L = 2, 16, 16


@functools.lru_cache(maxsize=None)
def _make_sc(B, S, D, nt):
    NW = _NC * _NS
    wpw = S // NW
    CH = 4
    CE = CH * D
    npc = wpw // CH
    npt = B * npc
    NB = 8
    K = 4
    cshift = (D // _L).bit_length() - 1
    cmask = (D // _L) - 1
    pshift = npc.bit_length() - 1
    pmask = npc - 1

    mesh = plsc.VectorSubcoreMesh(core_axis_name="c", subcore_axis_name="s")
    out_t = jax.ShapeDtypeStruct((B * S, D), jnp.float32)

    @functools.partial(
        pl.kernel,
        mesh=mesh,
        out_type=[out_t] * nt,
        scratch_types=(
            [pltpu.VMEM((CH, D), jnp.float32) for _ in range(NB)]
            + [pltpu.VMEM((wpw, D), jnp.float32)]
            + [pltpu.SemaphoreType.DMA for _ in range(2 * NB)]
        ),
    )
    def k(*args):
        ins = args[:nt]
        p_hbm = args[nt]
        outs = args[nt + 1:2 * nt + 1]
        scratch = args[2 * nt + 1:]
        xb = scratch[:NB]
        pall = scratch[NB]
        in_s = scratch[NB + 1:NB + 1 + NB]
        out_s = scratch[NB + 1 + NB:]

        wid = lax.axis_index("s") * _NC + lax.axis_index("c")
        prow0 = wid * wpw

        def rbase(ci):
            b = lax.shift_right_logical(ci, pshift)
            pc = lax.bitwise_and(ci, pmask)
            return b * S + prow0 + pc * CH

        def prow(ci):
            return lax.bitwise_and(ci, pmask) * CH

        def compute(q, ci):
            xq = xb[q]
            pr = prow(ci)

            @plsc.parallel_loop(0, CE // _L, unroll=8)
            def _(i):
                r = lax.shift_right_logical(i, cshift)
                c = pl.multiple_of(
                    lax.shift_left(lax.bitwise_and(i, cmask), 4), _L)
                plsc.addupdate(xq.at[r, pl.ds(c, _L)],
                               pall[pr + r, pl.ds(c, _L)])

        pltpu.sync_copy(p_hbm.at[pl.ds(prow0, wpw)], pall)

        for t in range(nt):
            x_hbm, o_hbm = ins[t], outs[t]

            def issue_in(ci, q):
                pltpu.async_copy(x_hbm.at[pl.ds(rbase(ci), CH)],
                                 xb[q], in_s[q])

            def wait_in(q):
                pltpu.make_async_copy(
                    x_hbm.at[pl.ds(0, CH)], xb[q], in_s[q]).wait()

            def issue_out(ci, q):
                pltpu.async_copy(xb[q], o_hbm.at[pl.ds(rbase(ci), CH)],
                                 out_s[q])

            def wait_out(q):
                pltpu.make_async_copy(
                    xb[q], o_hbm.at[pl.ds(0, CH)], out_s[q]).wait()

            for q in range(K):
                issue_in(q, q)

            n_it = npt // NB

            @pl.loop(0, n_it)
            def _(j):
                for q in range(NB):
                    ci = j * NB + q
                    wait_in(q)
                    compute(q, ci)
                    issue_out(ci, q)
                    nq = (q + K) % NB
                    if q < NB - K:
                        @pl.when(j > 0)
                        def _():
                            wait_out(nq)
                        issue_in(ci + K, nq)
                    else:
                        @pl.when(j < n_it - 1)
                        def _():
                            wait_out(nq)
                            issue_in(ci + K, nq)

            for q in range(NB):
                wait_out(q)

    return k


def _tc_body(x0_ref, x1_ref, pos_ref, o0, o1):
    p = pos_ref[...]
    o0[0] = x0_ref[0] + p
    o1[0] = x1_ref[0] + p


@functools.lru_cache(maxsize=None)
def _make_tc(B, S, D):
    BS = 512
    x_spec = pl.BlockSpec((1, BS, D), lambda s, b: (b, s, 0))
    pos_spec = pl.BlockSpec((BS, D), lambda s, b: (s, 0))
    out_shape = jax.ShapeDtypeStruct((B, S, D), jnp.float32)
    return pl.pallas_call(
        _tc_body,
        grid=(S // BS, B),
        in_specs=[x_spec, x_spec, pos_spec],
        out_specs=[x_spec, x_spec],
        out_shape=[out_shape, out_shape],
    )


def kernel(rtg, state, action, pos_table):
    B, S, D = rtg.shape
    pos = pos_table[:S]
    (o2,) = _make_sc(B, S, D, 1)(action.reshape(B * S, D), pos)
    o0, o1 = _make_tc(B, S, D)(rtg, state, pos)
    return (o0, o1, o2.reshape(B, S, D))


# speedup vs baseline: 6.0453x; 1.3233x over previous
import functools

import jax
import jax.numpy as jnp
from jax import lax
from jax.experimental import pallas as pl
from jax.experimental.pallas import tpu as pltpu
from jax.experimental.pallas import tpu_sc as plsc

_NC, _NS, _L = 2, 16, 16


@functools.lru_cache(maxsize=None)
def _make_sc(B, S, D, nt):
    NW = _NC * _NS
    wpw = S // NW
    CH = 4
    CE = CH * D
    npc = wpw // CH
    npt = B * npc
    NB = 8
    K = 4
    cshift = (D // _L).bit_length() - 1
    cmask = (D // _L) - 1
    pshift = npc.bit_length() - 1
    pmask = npc - 1

    mesh = plsc.VectorSubcoreMesh(core_axis_name="c", subcore_axis_name="s")
    out_t = jax.ShapeDtypeStruct((B * S, D), jnp.float32)

    @functools.partial(
        pl.kernel,
        mesh=mesh,
        out_type=[out_t] * nt,
        scratch_types=(
            [pltpu.VMEM((CH, D), jnp.float32) for _ in range(NB)]
            + [pltpu.VMEM((wpw, D), jnp.float32)]
            + [pltpu.SemaphoreType.DMA for _ in range(2 * NB)]
        ),
    )
    def k(*args):
        ins = args[:nt]
        p_hbm = args[nt]
        outs = args[nt + 1:2 * nt + 1]
        scratch = args[2 * nt + 1:]
        xb = scratch[:NB]
        pall = scratch[NB]
        in_s = scratch[NB + 1:NB + 1 + NB]
        out_s = scratch[NB + 1 + NB:]

        wid = lax.axis_index("s") * _NC + lax.axis_index("c")
        prow0 = wid * wpw

        def rbase(ci):
            b = lax.shift_right_logical(ci, pshift)
            pc = lax.bitwise_and(ci, pmask)
            return b * S + prow0 + pc * CH

        def prow(ci):
            return lax.bitwise_and(ci, pmask) * CH

        def compute(q, ci):
            xq = xb[q]
            pr = prow(ci)

            @plsc.parallel_loop(0, CE // _L, unroll=8)
            def _(i):
                r = lax.shift_right_logical(i, cshift)
                c = pl.multiple_of(
                    lax.shift_left(lax.bitwise_and(i, cmask), 4), _L)
                plsc.addupdate(xq.at[r, pl.ds(c, _L)],
                               pall[pr + r, pl.ds(c, _L)])

        pltpu.sync_copy(p_hbm.at[pl.ds(prow0, wpw)], pall)

        for t in range(nt):
            x_hbm, o_hbm = ins[t], outs[t]

            def issue_in(ci, q):
                pltpu.async_copy(x_hbm.at[pl.ds(rbase(ci), CH)],
                                 xb[q], in_s[q])

            def wait_in(q):
                pltpu.make_async_copy(
                    x_hbm.at[pl.ds(0, CH)], xb[q], in_s[q]).wait()

            def issue_out(ci, q):
                pltpu.async_copy(xb[q], o_hbm.at[pl.ds(rbase(ci), CH)],
                                 out_s[q])

            def wait_out(q):
                pltpu.make_async_copy(
                    xb[q], o_hbm.at[pl.ds(0, CH)], out_s[q]).wait()

            for q in range(K):
                issue_in(q, q)

            n_it = npt // NB

            @pl.loop(0, n_it)
            def _(j):
                for q in range(NB):
                    ci = j * NB + q
                    wait_in(q)
                    compute(q, ci)
                    issue_out(ci, q)
                    nq = (q + K) % NB
                    if q < NB - K:
                        @pl.when(j > 0)
                        def _():
                            wait_out(nq)
                        issue_in(ci + K, nq)
                    else:
                        @pl.when(j < n_it - 1)
                        def _():
                            wait_out(nq)
                            issue_in(ci + K, nq)

            for q in range(NB):
                wait_out(q)

    return k


def _tc_body2(x0_ref, x1_ref, pos_ref, o0, o1):
    p = pos_ref[...]
    o0[0] = x0_ref[0] + p
    o1[0] = x1_ref[0] + p


def _tc_body3(x0_ref, x1_ref, x2_ref, pos_ref, o0, o1, o2):
    p = pos_ref[...]
    o0[0] = x0_ref[0] + p
    o1[0] = x1_ref[0] + p
    o2[0] = x2_ref[0] + p


@functools.lru_cache(maxsize=None)
def _make_tc(B, S, D, nt, BS=512):
    x_spec = pl.BlockSpec((1, BS, D), lambda s, b: (b, s, 0))
    pos_spec = pl.BlockSpec((BS, D), lambda s, b: (s, 0))
    out_shape = jax.ShapeDtypeStruct((B, S, D), jnp.float32)
    body = {2: _tc_body2, 3: _tc_body3}[nt]
    return pl.pallas_call(
        body,
        grid=(S // BS, B),
        in_specs=[x_spec] * nt + [pos_spec],
        out_specs=[x_spec] * nt,
        out_shape=[out_shape] * nt,
    )


def kernel(rtg, state, action, pos_table):
    B, S, D = rtg.shape
    pos = pos_table[:S]
    o0, o1, o2 = _make_tc(B, S, D, 3, BS=1024)(rtg, state, action, pos)
    return (o0, o1, o2)
